# baseline (speedup 1.0000x reference)
import jax
import jax.numpy as jnp
from jax import lax
from jax.experimental import pallas as pl
from jax.experimental.pallas import tpu as pltpu
from jax.experimental.pallas import tpu_sc as plsc

NC = 2
NS = 16
NW = NC * NS
L = 16

ROWS = 500000
COLS = 80
TC_ROWS = 435200
SC_ROWS = ROWS - TC_ROWS
TC_BLOCK = 1024
CHUNK_ROWS = 400
RU = 4
NCHUNKS = SC_ROWS // CHUNK_ROWS
NSEG = COLS // L


def _body(scores_hbm, thr_hbm, out_hbm, bufs, obufs, thr_v, sems, osems):
    wid = lax.axis_index("s") * NC + lax.axis_index("c")
    pltpu.sync_copy(thr_hbm, thr_v)
    thr_s = thr_v[...][0]
    iotas = [lax.iota(jnp.int32, L) + L * t for t in range(NSEG)]
    lane0 = lax.iota(jnp.int32, L) == 0

    nw = (NCHUNKS - wid + NW - 1) // NW

    def chunk_row0(i):
        return (wid + i * NW) * CHUNK_ROWS

    def start_in(i, b):
        pltpu.async_copy(
            scores_hbm.at[pl.ds(TC_ROWS + chunk_row0(i), CHUNK_ROWS), :],
            bufs[b], sems[b])

    def wait_in(b):
        pltpu.make_async_copy(
            scores_hbm.at[pl.ds(0, CHUNK_ROWS), :], bufs[b], sems[b]).wait()

    def start_out(i, b):
        pltpu.async_copy(
            obufs[b], out_hbm.at[pl.ds(chunk_row0(i), CHUNK_ROWS)], osems[b])

    def wait_out(b):
        pltpu.make_async_copy(
            obufs[b], out_hbm.at[pl.ds(0, CHUNK_ROWS)], osems[b]).wait()

    def process(buf, obuf):
        @pl.loop(0, CHUNK_ROWS // RU)
        def _g(g):
            for j in range(RU):
                r = g * RU + j
                vs = [buf[r, pl.ds(L * t, L)] for t in range(NSEG)]
                m01 = jnp.maximum(vs[0], vs[1])
                m23 = jnp.maximum(vs[2], vs[3])
                m = jnp.maximum(jnp.maximum(m01, m23), vs[4])
                rowmax = jnp.max(m)
                cand = jnp.where(vs[0] == rowmax, iotas[0], jnp.int32(127))
                for t in range(1, NSEG):
                    cand = jnp.minimum(
                        cand,
                        jnp.where(vs[t] == rowmax, iotas[t], jnp.int32(127)))
                idx = jnp.min(cand)
                cid = jnp.where(rowmax < thr_s, 0, idx)
                plsc.store_scatter(
                    obuf, [jnp.full((L,), r, jnp.int32)],
                    jnp.full((L,), cid, jnp.int32), mask=lane0)

    start_in(0, 0)

    @pl.loop(0, (nw + 1) // 2)
    def _super(s):
        i0 = 2 * s
        i1 = i0 + 1

        @pl.when(i1 < nw)
        def _():
            start_in(i1, 1)
        wait_in(0)

        @pl.when(s > 0)
        def _():
            wait_out(0)
        process(bufs[0], obufs[0])
        start_out(i0, 0)

        @pl.when(i1 < nw)
        def _():
            @pl.when(i0 + 2 < nw)
            def _():
                start_in(i0 + 2, 0)
            wait_in(1)

            @pl.when(s > 0)
            def _():
                wait_out(1)
            process(bufs[1], obufs[1])
            start_out(i1, 1)

    wait_out(0)
    wait_out(1)


def _tc_body(thr_ref, x_ref, o_ref):
    x = x_ref[...]
    m = jnp.max(x, axis=1)
    iota = lax.broadcasted_iota(jnp.int32, (TC_BLOCK, COLS), 1)
    eq = x == m[:, None]
    idx = jnp.min(jnp.where(eq, iota, COLS), axis=1)
    o_ref[...] = jnp.where(m < thr_ref[0], 0, idx)


def _tc_kernel(scores, score_threshold):
    thr = jnp.full((1,), score_threshold, jnp.float32)
    return pl.pallas_call(
        _tc_body,
        grid=(TC_ROWS // TC_BLOCK,),
        in_specs=[
            pl.BlockSpec(memory_space=pltpu.SMEM),
            pl.BlockSpec((TC_BLOCK, COLS), lambda i: (i, 0)),
        ],
        out_specs=pl.BlockSpec((TC_BLOCK,), lambda i: (i,)),
        out_shape=jax.ShapeDtypeStruct((TC_ROWS,), jnp.int32),
        compiler_params=pltpu.CompilerParams(
            dimension_semantics=("arbitrary",)),
    )(thr, scores)


def kernel(scores, score_threshold):
    thr_vec = jnp.full((L,), score_threshold, jnp.float32)
    mesh = plsc.VectorSubcoreMesh(core_axis_name="c", subcore_axis_name="s")
    k = pl.kernel(
        _body,
        out_type=jax.ShapeDtypeStruct((SC_ROWS,), jnp.int32),
        mesh=mesh,
        compiler_params=pltpu.CompilerParams(
            needs_layout_passes=False, use_tc_tiling_on_sc=True),
        scratch_types=[
            [pltpu.VMEM((CHUNK_ROWS, COLS), jnp.float32) for _ in range(2)],
            [pltpu.VMEM((CHUNK_ROWS,), jnp.int32) for _ in range(2)],
            pltpu.VMEM((L,), jnp.float32),
            [pltpu.SemaphoreType.DMA for _ in range(2)],
            [pltpu.SemaphoreType.DMA for _ in range(2)],
        ],
    )
    sc_out = k(scores, thr_vec)
    tc_out = _tc_kernel(scores, score_threshold)
    return jnp.concatenate([tc_out, sc_out])


# speedup vs baseline: 3.7259x; 3.7259x over previous
import jax
import jax.numpy as jnp
from jax import lax
from jax.experimental import pallas as pl
from jax.experimental.pallas import tpu as pltpu
from jax.experimental.pallas import tpu_sc as plsc

NC = 2
NS = 16
NW = NC * NS
L = 16

ROWS = 500000
COLS = 80
SC_ROWS = 49152
TC_ROWS = ROWS - SC_ROWS
TC_BLOCK = 2048
CHUNK = 128
NCHUNKS = SC_ROWS // CHUNK
NCHAIN = 4


def _sc_body(scores_hbm, thr_hbm, out_hbm, bufs, obufs, thr_v, sems, osems):
    wid = lax.axis_index("s") * NC + lax.axis_index("c")
    pltpu.sync_copy(thr_hbm, thr_v)
    thr = thr_v[...]

    nw = (NCHUNKS - wid + NW - 1) // NW

    def chunk_row0(i):
        return (wid + i * NW) * CHUNK

    def start_in(i, b):
        pltpu.async_copy(
            scores_hbm.at[:, pl.ds(chunk_row0(i), CHUNK)],
            bufs[b], sems[b])

    def wait_in(b):
        pltpu.make_async_copy(
            scores_hbm.at[:, pl.ds(0, CHUNK)], bufs[b], sems[b]).wait()

    def start_out(i, b):
        pltpu.async_copy(
            obufs[b], out_hbm.at[pl.ds(chunk_row0(i), CHUNK)], osems[b])

    def wait_out(b):
        pltpu.make_async_copy(
            obufs[b], out_hbm.at[pl.ds(0, CHUNK)], osems[b]).wait()

    def process(buf, obuf):
        @pl.loop(0, CHUNK // L)
        def _group(g):
            base = g * L
            vmaxs = []
            vidxs = []
            for k in range(NCHAIN):
                vmaxs.append(buf[k, pl.ds(base, L)])
                vidxs.append(jnp.full((L,), k, jnp.int32))
            for cc in range(1, COLS // NCHAIN):
                for k in range(NCHAIN):
                    c = cc * NCHAIN + k
                    v = buf[c, pl.ds(base, L)]
                    gt = v > vmaxs[k]
                    vidxs[k] = jnp.where(gt, jnp.int32(c), vidxs[k])
                    vmaxs[k] = jnp.maximum(vmaxs[k], v)
            m, ix = vmaxs[0], vidxs[0]
            for k in range(1, NCHAIN):
                b, bix = vmaxs[k], vidxs[k]
                take = (b > m) | ((b == m) & (bix < ix))
                m = jnp.where(take, b, m)
                ix = jnp.where(take, bix, ix)
            ix = jnp.where(m < thr, 0, ix)
            obuf[pl.ds(base, L)] = ix

    start_in(0, 0)

    @pl.loop(0, (nw + 1) // 2)
    def _super(s):
        i0 = 2 * s
        i1 = i0 + 1

        @pl.when(i1 < nw)
        def _():
            start_in(i1, 1)
        wait_in(0)

        @pl.when(s > 0)
        def _():
            wait_out(0)
        process(bufs[0], obufs[0])
        start_out(i0, 0)

        @pl.when(i1 < nw)
        def _():
            @pl.when(i0 + 2 < nw)
            def _():
                start_in(i0 + 2, 0)
            wait_in(1)

            @pl.when(s > 0)
            def _():
                wait_out(1)
            process(bufs[1], obufs[1])
            start_out(i1, 1)

    wait_out(0)
    wait_out(1)


def _sc_kernel(scores_t, score_threshold):
    thr_vec = jnp.full((L,), score_threshold, jnp.float32)
    mesh = plsc.VectorSubcoreMesh(core_axis_name="c", subcore_axis_name="s")
    k = pl.kernel(
        _sc_body,
        out_type=jax.ShapeDtypeStruct((SC_ROWS,), jnp.int32),
        mesh=mesh,
        compiler_params=pltpu.CompilerParams(
            needs_layout_passes=False, use_tc_tiling_on_sc=True),
        scratch_types=[
            [pltpu.VMEM((COLS, CHUNK), jnp.float32) for _ in range(2)],
            [pltpu.VMEM((CHUNK,), jnp.int32) for _ in range(2)],
            pltpu.VMEM((L,), jnp.float32),
            [pltpu.SemaphoreType.DMA for _ in range(2)],
            [pltpu.SemaphoreType.DMA for _ in range(2)],
        ],
    )
    return k(scores_t, thr_vec)


def _tc_body(thr_ref, x_ref, o_ref):
    x = x_ref[...]
    m = jnp.max(x, axis=0)
    iota = lax.broadcasted_iota(jnp.int32, (COLS, TC_BLOCK), 0)
    eq = x == m[None, :]
    idx = jnp.min(jnp.where(eq, iota, COLS), axis=0)
    o_ref[...] = jnp.where(m < thr_ref[0], 0, idx)


def _tc_kernel(scores_t, score_threshold):
    thr = jnp.full((1,), score_threshold, jnp.float32)
    grid = (TC_ROWS + TC_BLOCK - 1) // TC_BLOCK
    return pl.pallas_call(
        _tc_body,
        grid=(grid,),
        in_specs=[
            pl.BlockSpec(memory_space=pltpu.SMEM),
            pl.BlockSpec((COLS, TC_BLOCK),
                         lambda i: (0, SC_ROWS // TC_BLOCK + i)),
        ],
        out_specs=pl.BlockSpec((TC_BLOCK,), lambda i: (i,)),
        out_shape=jax.ShapeDtypeStruct((TC_ROWS,), jnp.int32),
        compiler_params=pltpu.CompilerParams(
            dimension_semantics=("arbitrary",)),
    )(thr, scores_t)


def kernel(scores, score_threshold):
    scores_t = scores.T
    sc_out = _sc_kernel(scores_t, score_threshold)
    tc_out = _tc_kernel(scores_t, score_threshold)
    return jnp.concatenate([sc_out, tc_out])


# speedup vs baseline: 6.8563x; 1.8401x over previous
import jax
import jax.numpy as jnp
from jax import lax
from jax.experimental import pallas as pl
from jax.experimental.pallas import tpu as pltpu
from jax.experimental.pallas import tpu_sc as plsc

NC = 2
NS = 16
NW = NC * NS
L = 16

ROWS = 500000
COLS = 80
SC_ROWS = 49152
TC_ROWS = ROWS - SC_ROWS
TC_BLOCK = 8192
CHUNK = 128
NCHUNKS = SC_ROWS // CHUNK
NCHAIN = 4


def _sc_body(scores_hbm, thr_hbm, out_hbm, bufs, obufs, thr_v, sems, osems):
    wid = lax.axis_index("s") * NC + lax.axis_index("c")
    pltpu.sync_copy(thr_hbm, thr_v)
    thr = thr_v[...]

    nw = (NCHUNKS - wid + NW - 1) // NW

    def chunk_row0(i):
        return (wid + i * NW) * CHUNK

    def start_in(i, b):
        pltpu.async_copy(
            scores_hbm.at[:, pl.ds(chunk_row0(i), CHUNK)],
            bufs[b], sems[b])

    def wait_in(b):
        pltpu.make_async_copy(
            scores_hbm.at[:, pl.ds(0, CHUNK)], bufs[b], sems[b]).wait()

    def start_out(i, b):
        pltpu.async_copy(
            obufs[b], out_hbm.at[pl.ds(chunk_row0(i), CHUNK)], osems[b])

    def wait_out(b):
        pltpu.make_async_copy(
            obufs[b], out_hbm.at[pl.ds(0, CHUNK)], osems[b]).wait()

    def process(buf, obuf):
        @pl.loop(0, CHUNK // L)
        def _group(g):
            base = g * L
            vmaxs = []
            vidxs = []
            for k in range(NCHAIN):
                vmaxs.append(buf[k, pl.ds(base, L)])
                vidxs.append(jnp.full((L,), k, jnp.int32))
            for cc in range(1, COLS // NCHAIN):
                for k in range(NCHAIN):
                    c = cc * NCHAIN + k
                    v = buf[c, pl.ds(base, L)]
                    gt = v > vmaxs[k]
                    vidxs[k] = jnp.where(gt, jnp.int32(c), vidxs[k])
                    vmaxs[k] = jnp.maximum(vmaxs[k], v)
            m, ix = vmaxs[0], vidxs[0]
            for k in range(1, NCHAIN):
                b, bix = vmaxs[k], vidxs[k]
                take = (b > m) | ((b == m) & (bix < ix))
                m = jnp.where(take, b, m)
                ix = jnp.where(take, bix, ix)
            ix = jnp.where(m < thr, 0, ix)
            obuf[pl.ds(base, L)] = ix

    start_in(0, 0)

    @pl.loop(0, (nw + 1) // 2)
    def _super(s):
        i0 = 2 * s
        i1 = i0 + 1

        @pl.when(i1 < nw)
        def _():
            start_in(i1, 1)
        wait_in(0)

        @pl.when(s > 0)
        def _():
            wait_out(0)
        process(bufs[0], obufs[0])
        start_out(i0, 0)

        @pl.when(i1 < nw)
        def _():
            @pl.when(i0 + 2 < nw)
            def _():
                start_in(i0 + 2, 0)
            wait_in(1)

            @pl.when(s > 0)
            def _():
                wait_out(1)
            process(bufs[1], obufs[1])
            start_out(i1, 1)

    wait_out(0)
    wait_out(1)


def _sc_kernel(scores_t, score_threshold):
    thr_vec = jnp.full((L,), score_threshold, jnp.float32)
    mesh = plsc.VectorSubcoreMesh(core_axis_name="c", subcore_axis_name="s")
    k = pl.kernel(
        _sc_body,
        out_type=jax.ShapeDtypeStruct((SC_ROWS,), jnp.int32),
        mesh=mesh,
        compiler_params=pltpu.CompilerParams(
            needs_layout_passes=False, use_tc_tiling_on_sc=True),
        scratch_types=[
            [pltpu.VMEM((COLS, CHUNK), jnp.float32) for _ in range(2)],
            [pltpu.VMEM((CHUNK,), jnp.int32) for _ in range(2)],
            pltpu.VMEM((L,), jnp.float32),
            [pltpu.SemaphoreType.DMA for _ in range(2)],
            [pltpu.SemaphoreType.DMA for _ in range(2)],
        ],
    )
    return k(scores_t, thr_vec)


def _tc_body(thr_ref, x_ref, o_ref):
    x = x_ref[...]
    m = jnp.max(x, axis=0)
    iota = lax.broadcasted_iota(jnp.int32, (COLS, TC_BLOCK), 0)
    eq = x == m[None, :]
    idx = jnp.min(jnp.where(eq, iota, COLS), axis=0)
    o_ref[...] = jnp.where(m < thr_ref[0], 0, idx)


def _tc_kernel(scores_t, score_threshold):
    thr = jnp.full((1,), score_threshold, jnp.float32)
    grid = (TC_ROWS + TC_BLOCK - 1) // TC_BLOCK
    return pl.pallas_call(
        _tc_body,
        grid=(grid,),
        in_specs=[
            pl.BlockSpec(memory_space=pltpu.SMEM),
            pl.BlockSpec((COLS, TC_BLOCK),
                         lambda i: (0, SC_ROWS // TC_BLOCK + i)),
        ],
        out_specs=pl.BlockSpec((TC_BLOCK,), lambda i: (i,)),
        out_shape=jax.ShapeDtypeStruct((TC_ROWS,), jnp.int32),
        compiler_params=pltpu.CompilerParams(
            dimension_semantics=("arbitrary",)),
    )(thr, scores_t)


def kernel(scores, score_threshold):
    scores_t = scores.T
    sc_out = _sc_kernel(scores_t, score_threshold)
    tc_out = _tc_kernel(scores_t, score_threshold)
    return jnp.concatenate([sc_out, tc_out])


# speedup vs baseline: 8.3156x; 1.2129x over previous
import jax
import jax.numpy as jnp
from jax import lax
from jax.experimental import pallas as pl
from jax.experimental.pallas import tpu as pltpu
from jax.experimental.pallas import tpu_sc as plsc

NC = 2
NS = 16
NW = NC * NS
L = 16

ROWS = 500000
COLS = 80
SC_ROWS = 196608
TC_ROWS = ROWS - SC_ROWS
TC_BLOCK = 16384
CHUNK = 128
NCHUNKS = SC_ROWS // CHUNK
NCHAIN = 4


def _sc_body(scores_hbm, thr_hbm, out_hbm, bufs, obufs, thr_v, sems, osems):
    wid = lax.axis_index("s") * NC + lax.axis_index("c")
    pltpu.sync_copy(thr_hbm, thr_v)
    thr = thr_v[...]

    nw = (NCHUNKS - wid + NW - 1) // NW

    def chunk_row0(i):
        return (wid + i * NW) * CHUNK

    def start_in(i, b):
        pltpu.async_copy(
            scores_hbm.at[:, pl.ds(chunk_row0(i), CHUNK)],
            bufs[b], sems[b])

    def wait_in(b):
        pltpu.make_async_copy(
            scores_hbm.at[:, pl.ds(0, CHUNK)], bufs[b], sems[b]).wait()

    def start_out(i, b):
        pltpu.async_copy(
            obufs[b], out_hbm.at[pl.ds(chunk_row0(i), CHUNK)], osems[b])

    def wait_out(b):
        pltpu.make_async_copy(
            obufs[b], out_hbm.at[pl.ds(0, CHUNK)], osems[b]).wait()

    def process(buf, obuf):
        @pl.loop(0, CHUNK // L)
        def _group(g):
            base = g * L
            vmaxs = []
            vidxs = []
            for k in range(NCHAIN):
                vmaxs.append(buf[k, pl.ds(base, L)])
                vidxs.append(jnp.full((L,), k, jnp.int32))
            for cc in range(1, COLS // NCHAIN):
                for k in range(NCHAIN):
                    c = cc * NCHAIN + k
                    v = buf[c, pl.ds(base, L)]
                    gt = v > vmaxs[k]
                    vidxs[k] = jnp.where(gt, jnp.int32(c), vidxs[k])
                    vmaxs[k] = jnp.maximum(vmaxs[k], v)
            m, ix = vmaxs[0], vidxs[0]
            for k in range(1, NCHAIN):
                b, bix = vmaxs[k], vidxs[k]
                take = (b > m) | ((b == m) & (bix < ix))
                m = jnp.where(take, b, m)
                ix = jnp.where(take, bix, ix)
            ix = jnp.where(m < thr, 0, ix)
            obuf[pl.ds(base, L)] = ix

    start_in(0, 0)

    @pl.loop(0, (nw + 1) // 2)
    def _super(s):
        i0 = 2 * s
        i1 = i0 + 1

        @pl.when(i1 < nw)
        def _():
            start_in(i1, 1)
        wait_in(0)

        @pl.when(s > 0)
        def _():
            wait_out(0)
        process(bufs[0], obufs[0])
        start_out(i0, 0)

        @pl.when(i1 < nw)
        def _():
            @pl.when(i0 + 2 < nw)
            def _():
                start_in(i0 + 2, 0)
            wait_in(1)

            @pl.when(s > 0)
            def _():
                wait_out(1)
            process(bufs[1], obufs[1])
            start_out(i1, 1)

    wait_out(0)
    wait_out(1)


def _sc_kernel(scores_t, score_threshold):
    thr_vec = jnp.full((L,), score_threshold, jnp.float32)
    mesh = plsc.VectorSubcoreMesh(core_axis_name="c", subcore_axis_name="s")
    k = pl.kernel(
        _sc_body,
        out_type=jax.ShapeDtypeStruct((SC_ROWS,), jnp.int32),
        mesh=mesh,
        compiler_params=pltpu.CompilerParams(
            needs_layout_passes=False, use_tc_tiling_on_sc=True),
        scratch_types=[
            [pltpu.VMEM((COLS, CHUNK), jnp.float32) for _ in range(2)],
            [pltpu.VMEM((CHUNK,), jnp.int32) for _ in range(2)],
            pltpu.VMEM((L,), jnp.float32),
            [pltpu.SemaphoreType.DMA for _ in range(2)],
            [pltpu.SemaphoreType.DMA for _ in range(2)],
        ],
    )
    return k(scores_t, thr_vec)


def _tc_body(thr_ref, x_ref, o_ref):
    x = x_ref[...]
    m = jnp.max(x, axis=0)
    iota = lax.broadcasted_iota(jnp.int32, (COLS, TC_BLOCK), 0)
    eq = x == m[None, :]
    idx = jnp.min(jnp.where(eq, iota, COLS), axis=0)
    o_ref[...] = jnp.where(m < thr_ref[0], 0, idx)


def _tc_kernel(scores_t, score_threshold):
    thr = jnp.full((1,), score_threshold, jnp.float32)
    grid = (TC_ROWS + TC_BLOCK - 1) // TC_BLOCK
    return pl.pallas_call(
        _tc_body,
        grid=(grid,),
        in_specs=[
            pl.BlockSpec(memory_space=pltpu.SMEM),
            pl.BlockSpec((COLS, TC_BLOCK),
                         lambda i: (0, SC_ROWS // TC_BLOCK + i)),
        ],
        out_specs=pl.BlockSpec((TC_BLOCK,), lambda i: (i,)),
        out_shape=jax.ShapeDtypeStruct((TC_ROWS,), jnp.int32),
        compiler_params=pltpu.CompilerParams(
            dimension_semantics=("arbitrary",)),
    )(thr, scores_t)


def kernel(scores, score_threshold):
    scores_t = scores.T
    sc_out = _sc_kernel(scores_t, score_threshold)
    tc_out = _tc_kernel(scores_t, score_threshold)
    return jnp.concatenate([sc_out, tc_out])
